# baseline (speedup 1.0000x reference)
import jax
import jax.numpy as jnp
from jax import lax
from jax.experimental import pallas as pl
from jax.experimental.pallas import tpu as pltpu
from jax.experimental.pallas import tpu_sc as plsc

_NUM_TOKENS = 32768
_D_MODEL = 1024
_NUM_EXPERTS = 64
_LANES = 16
_NUM_WORKERS = 16
_CHUNK = _NUM_TOKENS // _NUM_WORKERS
_VECS = _CHUNK // _LANES
_GRID = 16
_BLOCK_ROWS = _NUM_TOKENS // _GRID


def _hist_body(tag_hbm, load_hbm, out_hbm, tag_v, hist_v, red_v, buf_v, load_v,
               shared):
    sid = lax.axis_index("s")
    lanes = lax.iota(jnp.int32, _LANES)
    zeros = jnp.zeros((_LANES,), jnp.int32)
    ones = jnp.ones((_LANES,), jnp.int32)

    for b in range(_NUM_EXPERTS):
        hist_v[pl.ds(b * _LANES, _LANES)] = zeros

    pltpu.sync_copy(tag_hbm.at[pl.ds(sid * _CHUNK, _CHUNK)], tag_v)

    def body(i, carry):
        t = tag_v[pl.ds(i * _LANES, _LANES)]
        plsc.addupdate_scatter(hist_v, [t * _LANES + lanes], ones)
        return carry

    lax.fori_loop(0, _VECS, body, 0)

    for k in range(_NUM_EXPERTS // _LANES):
        rows = (lanes + (k * _LANES)) * _LANES
        acc = plsc.load_gather(hist_v, [rows])
        for c in range(1, _LANES):
            acc = acc + plsc.load_gather(hist_v, [rows + c])
        red_v[pl.ds(k * _LANES, _LANES)] = acc

    pltpu.sync_copy(red_v, shared.at[pl.ds(sid * _NUM_EXPERTS, _NUM_EXPERTS)])
    plsc.subcore_barrier()

    @pl.when(sid == 0)
    def _():
        pltpu.sync_copy(load_hbm, load_v)
        pltpu.sync_copy(shared, buf_v)
        lv = load_v[...]
        for k in range(_NUM_EXPERTS // _LANES):
            acc = buf_v[pl.ds(k * _LANES, _LANES)]
            for w in range(1, _NUM_WORKERS):
                acc = acc + buf_v[pl.ds(w * _NUM_EXPERTS + k * _LANES, _LANES)]
            red_v[pl.ds(k * _LANES, _LANES)] = jnp.minimum(acc, lv)
        pltpu.sync_copy(red_v, out_hbm)


def _capacity_sc(tag, load_vec):
    mesh = plsc.VectorSubcoreMesh(
        core_axis_name="c", subcore_axis_name="s",
        num_cores=1, num_subcores=_NUM_WORKERS)
    return pl.kernel(
        _hist_body,
        out_type=jax.ShapeDtypeStruct((_NUM_EXPERTS,), jnp.int32),
        mesh=mesh,
        compiler_params=pltpu.CompilerParams(needs_layout_passes=False),
        scratch_types=[
            pltpu.VMEM((_CHUNK,), jnp.int32),
            pltpu.VMEM((_NUM_EXPERTS * _LANES,), jnp.int32),
            pltpu.VMEM((_NUM_EXPERTS,), jnp.int32),
            pltpu.VMEM((_NUM_WORKERS * _NUM_EXPERTS,), jnp.int32),
            pltpu.VMEM((_LANES,), jnp.int32),
            pltpu.VMEM_SHARED((_NUM_WORKERS * _NUM_EXPERTS,), jnp.int32),
        ],
    )(tag, load_vec)


@jax.jit
def _annotate(x, tag, load_vec):
    capacity = _capacity_sc(tag, load_vec)
    scale = (load_vec[0] * 0 + 1).astype(jnp.float32)
    x_out = x * scale
    return x_out, capacity


def kernel(x, tag, load):
    load_vec = jnp.full((_LANES,), load, dtype=jnp.int32)
    x_out, capacity = _annotate(x, tag, load_vec)
    return (x_out, tag, capacity)


# speedup vs baseline: 1.2038x; 1.2038x over previous
import jax
import jax.numpy as jnp
from jax.experimental import pallas as pl
from jax.experimental.pallas import tpu as pltpu

_NUM_TOKENS = 32768
_D_MODEL = 1024
_NUM_EXPERTS = 64
_ROWS = 256
_COLS = 128
_GRID = 32
_BLOCK_ROWS = _NUM_TOKENS // _GRID


def _fused_body(x_ref, tag_ref, load_ref, xout_ref, cap_ref):
    @pl.when(pl.program_id(0) == 0)
    def _():
        tags = tag_ref[...]
        load = load_ref[0, 0]
        for e in range(_NUM_EXPERTS):
            cnt = jnp.sum(jnp.where(tags == e, 1, 0))
            cap_ref[e] = jnp.minimum(cnt, load)

    xout_ref[...] = x_ref[...]


@jax.jit
def _fused(x, tag2d, load_arr):
    return pl.pallas_call(
        _fused_body,
        grid=(_GRID,),
        in_specs=[
            pl.BlockSpec((_BLOCK_ROWS, _D_MODEL), lambda i: (i, 0)),
            pl.BlockSpec((_ROWS, _COLS), lambda i: (0, 0)),
            pl.BlockSpec(memory_space=pltpu.SMEM),
        ],
        out_specs=[
            pl.BlockSpec((_BLOCK_ROWS, _D_MODEL), lambda i: (i, 0)),
            pl.BlockSpec(memory_space=pltpu.SMEM),
        ],
        out_shape=[
            jax.ShapeDtypeStruct((_NUM_TOKENS, _D_MODEL), jnp.float32),
            jax.ShapeDtypeStruct((_NUM_EXPERTS,), jnp.int32),
        ],
    )(x, tag2d, load_arr)


def kernel(x, tag, load):
    tag2d = tag.reshape(_ROWS, _COLS)
    load_arr = jnp.full((1, 1), load, dtype=jnp.int32)
    x_out, capacity = _fused(x, tag2d, load_arr)
    return (x_out, tag, capacity)


# speedup vs baseline: 1.2362x; 1.0269x over previous
import jax
import jax.numpy as jnp
from jax.experimental import pallas as pl
from jax.experimental.pallas import tpu as pltpu

_NUM_TOKENS = 32768
_D_MODEL = 1024
_NUM_EXPERTS = 64
_ROWS = 256
_COLS = 128
_GRID = 16
_BLOCK_ROWS = _NUM_TOKENS // _GRID


def _fused_body(x_ref, tag_ref, load_ref, xout_ref, cap_ref):
    @pl.when(pl.program_id(0) == 0)
    def _():
        tags = tag_ref[...]
        load = load_ref[0, 0]
        for e in range(_NUM_EXPERTS):
            cnt = jnp.sum(jnp.where(tags == e, 1, 0))
            cap_ref[e] = jnp.minimum(cnt, load)

    xout_ref[...] = x_ref[...]


@jax.jit
def _fused(x, tag2d, load_arr):
    return pl.pallas_call(
        _fused_body,
        grid=(_GRID,),
        in_specs=[
            pl.BlockSpec((_BLOCK_ROWS, _D_MODEL), lambda i: (i, 0)),
            pl.BlockSpec((_ROWS, _COLS), lambda i: (0, 0)),
            pl.BlockSpec(memory_space=pltpu.SMEM),
        ],
        out_specs=[
            pl.BlockSpec((_BLOCK_ROWS, _D_MODEL), lambda i: (i, 0)),
            pl.BlockSpec(memory_space=pltpu.SMEM),
        ],
        out_shape=[
            jax.ShapeDtypeStruct((_NUM_TOKENS, _D_MODEL), jnp.float32),
            jax.ShapeDtypeStruct((_NUM_EXPERTS,), jnp.int32),
        ],
    )(x, tag2d, load_arr)


def kernel(x, tag, load):
    tag2d = tag.reshape(_ROWS, _COLS)
    load_arr = jnp.full((1, 1), load, dtype=jnp.int32)
    x_out, capacity = _fused(x, tag2d, load_arr)
    return (x_out, tag, capacity)


# speedup vs baseline: 1.2447x; 1.0069x over previous
import jax
import jax.numpy as jnp
from jax.experimental import pallas as pl
from jax.experimental.pallas import tpu as pltpu

_NUM_TOKENS = 32768
_D_MODEL = 1024
_NUM_EXPERTS = 64
_ROWS = 256
_COLS = 128
_GRID = 16
_BLOCK_ROWS = _NUM_TOKENS // _GRID


def _fused_body(x_ref, tag_ref, load_ref, xout_ref, tagout_ref, cap_ref):
    @pl.when(pl.program_id(0) == 0)
    def _():
        tags = tag_ref[...]
        tagout_ref[...] = tags
        load = load_ref[0, 0]
        for e in range(_NUM_EXPERTS):
            cnt = jnp.sum(jnp.where(tags == e, 1, 0))
            cap_ref[e] = jnp.minimum(cnt, load)

    xout_ref[...] = x_ref[...]


@jax.jit
def _fused(x, tag2d, load_arr):
    return pl.pallas_call(
        _fused_body,
        grid=(_GRID,),
        in_specs=[
            pl.BlockSpec((_BLOCK_ROWS, _D_MODEL), lambda i: (i, 0)),
            pl.BlockSpec((_ROWS, _COLS), lambda i: (0, 0)),
            pl.BlockSpec(memory_space=pltpu.SMEM),
        ],
        out_specs=[
            pl.BlockSpec((_BLOCK_ROWS, _D_MODEL), lambda i: (i, 0)),
            pl.BlockSpec((_ROWS, _COLS), lambda i: (0, 0)),
            pl.BlockSpec(memory_space=pltpu.SMEM),
        ],
        out_shape=[
            jax.ShapeDtypeStruct((_NUM_TOKENS, _D_MODEL), jnp.float32),
            jax.ShapeDtypeStruct((_ROWS, _COLS), jnp.int32),
            jax.ShapeDtypeStruct((_NUM_EXPERTS,), jnp.int32),
        ],
    )(x, tag2d, load_arr)


def kernel(x, tag, load):
    tag2d = tag.reshape(_ROWS, _COLS)
    load_arr = jnp.full((1, 1), load, dtype=jnp.int32)
    x_out, tag_out, capacity = _fused(x, tag2d, load_arr)
    return (x_out, tag_out.reshape(_NUM_TOKENS), capacity)
